# baseline (speedup 1.0000x reference)
import functools

import jax
import jax.numpy as jnp
from jax import lax
from jax.experimental import pallas as pl
from jax.experimental.pallas import tpu as pltpu
from jax.experimental.pallas import tpu_sc as plsc

F = 26
V = 100000
D = 32
B = 16384

NC = 2
NS = 16
NW = NC * NS
ROWS_PER_W = B // NW
CHUNK = 64
NCHUNK = ROWS_PER_W // CHUNK
IDXN = CHUNK * F
PAT = 208
LANES = 16


def _body(ids_hbm, offs_hbm, tab_hbm, out_hbm, idx_v, offs_v, gat_v, out_v, sem):
  wid = lax.axis_index("s") * NC + lax.axis_index("c")
  base = wid * ROWS_PER_W

  pltpu.sync_copy(offs_hbm, offs_v)

  def chunk_body(k, carry):
    row0 = base + k * CHUNK
    pltpu.sync_copy(ids_hbm.at[pl.ds(row0 * F, IDXN)], idx_v)
    for r in range(IDXN // PAT):
      for t in range(PAT // LANES):
        sl = pl.ds(r * PAT + t * LANES, LANES)
        idx_v[sl] = idx_v[sl] + offs_v[pl.ds(t * LANES, LANES)]
    pltpu.async_copy(tab_hbm.at[idx_v], gat_v, sem).wait()
    def red_body(c, carry2):
      r0 = c * F
      acc0 = gat_v[r0, pl.ds(0, LANES)]
      acc1 = gat_v[r0, pl.ds(LANES, LANES)]
      for f in range(1, F):
        acc0 = acc0 + gat_v[r0 + f, pl.ds(0, LANES)]
        acc1 = acc1 + gat_v[r0 + f, pl.ds(LANES, LANES)]
      out_v[c, pl.ds(0, LANES)] = acc0
      out_v[c, pl.ds(LANES, LANES)] = acc1
      return carry2

    lax.fori_loop(0, CHUNK, red_body, 0)
    pltpu.sync_copy(out_v, out_hbm.at[pl.ds(row0, CHUNK)])
    return carry

  lax.fori_loop(0, NCHUNK, chunk_body, 0)


@jax.jit
def kernel(ids, tables):
  ids_flat = ids.reshape(B * F)
  tab_flat = tables.reshape(F * V, D)
  offs = (jnp.arange(PAT, dtype=jnp.int32) % F) * V

  mesh = plsc.VectorSubcoreMesh(core_axis_name="c", subcore_axis_name="s")
  run = pl.kernel(
      _body,
      out_type=jax.ShapeDtypeStruct((B, D), jnp.float32),
      mesh=mesh,
      scratch_types=[
          pltpu.VMEM((IDXN,), jnp.int32),
          pltpu.VMEM((PAT,), jnp.int32),
          pltpu.VMEM((IDXN, D), jnp.float32),
          pltpu.VMEM((CHUNK, D), jnp.float32),
          pltpu.SemaphoreType.DMA,
      ],
      compiler_params=pltpu.CompilerParams(use_tc_tiling_on_sc=False),
  )
  return run(ids_flat, offs, tab_flat)


# speedup vs baseline: 3.1879x; 3.1879x over previous
import functools

import jax
import jax.numpy as jnp
from jax import lax
from jax.experimental import pallas as pl
from jax.experimental.pallas import tpu as pltpu
from jax.experimental.pallas import tpu_sc as plsc

F = 26
V = 100000
D = 32
B = 16384

NC = 2
NS = 16
NW = NC * NS
LANES = 16
IDS_CHUNK = 4096
NIDC = B // IDS_CHUNK
BLKS = IDS_CHUNK // LANES


def _body(ids_hbm, tab_hbm, out_hbm, col_v, ids0, ids1, acc_v, csem, isem0,
          isem1):
  c = lax.axis_index("s") * NC + lax.axis_index("c")
  ids_v = (ids0, ids1)
  isem = (isem0, isem1)

  def ids_start(f, j, p):
    pltpu.async_copy(ids_hbm.at[f, pl.ds(j * IDS_CHUNK, IDS_CHUNK)],
                     ids_v[p], isem[p])

  def ids_wait(f, j, p):
    pltpu.make_async_copy(ids_hbm.at[f, pl.ds(j * IDS_CHUNK, IDS_CHUNK)],
                          ids_v[p], isem[p]).wait()

  ids_start(0, 0, 0)
  for f in range(F):
    pltpu.sync_copy(tab_hbm.at[f, c], col_v)
    for j in range(NIDC):
      p = j % 2
      ids_wait(f, j, p)
      if j + 1 < NIDC:
        ids_start(f, j + 1, 1 - p)
      elif f + 1 < F:
        ids_start(f + 1, 0, 1 - p)
      idsb = ids_v[p]

      def blk_body(blk, carry, j=j, f=f, idsb=idsb):
        vidx = idsb[pl.ds(blk * LANES, LANES)]
        vals = plsc.load_gather(col_v, [vidx])
        sl = pl.ds(j * IDS_CHUNK + blk * LANES, LANES)
        if f == 0:
          acc_v[sl] = vals
        else:
          plsc.addupdate(acc_v.at[sl], vals)
        return carry

      lax.fori_loop(0, BLKS, blk_body, 0, unroll=4)

  pltpu.sync_copy(acc_v, out_hbm.at[c])


@jax.jit
def kernel(ids, tables):
  ids_t = ids.T
  tab_t = tables.transpose(0, 2, 1)

  mesh = plsc.VectorSubcoreMesh(core_axis_name="c", subcore_axis_name="s")
  run = pl.kernel(
      _body,
      out_type=jax.ShapeDtypeStruct((D, B), jnp.float32),
      mesh=mesh,
      scratch_types=[
          pltpu.VMEM((V,), jnp.float32),
          pltpu.VMEM((IDS_CHUNK,), jnp.int32),
          pltpu.VMEM((IDS_CHUNK,), jnp.int32),
          pltpu.VMEM((B,), jnp.float32),
          pltpu.SemaphoreType.DMA,
          pltpu.SemaphoreType.DMA,
          pltpu.SemaphoreType.DMA,
      ],
      compiler_params=pltpu.CompilerParams(needs_layout_passes=False),
  )
  return run(ids_t, tab_t).T


# speedup vs baseline: 5.2320x; 1.6412x over previous
import functools

import jax
import jax.numpy as jnp
from jax import lax
from jax.experimental import pallas as pl
from jax.experimental.pallas import tpu as pltpu
from jax.experimental.pallas import tpu_sc as plsc

F = 26
V = 100000
D = 32
B = 16384

NC = 2
NS = 16
NW = NC * NS
LANES = 16
IDS_CHUNK = 4096
NIDC = B // IDS_CHUNK
BLKS = IDS_CHUNK // LANES


def _body(ids_hbm, tab_hbm, out_hbm, col_v, ids0, ids1, acc_v, csem, isem0,
          isem1):
  c = lax.axis_index("s") * NC + lax.axis_index("c")
  ids_v = (ids0, ids1)
  isem = (isem0, isem1)

  def ids_start(f, j, p):
    pltpu.async_copy(ids_hbm.at[f, pl.ds(j * IDS_CHUNK, IDS_CHUNK)],
                     ids_v[p], isem[p])

  def ids_wait(f, j, p):
    pltpu.make_async_copy(ids_hbm.at[f, pl.ds(j * IDS_CHUNK, IDS_CHUNK)],
                          ids_v[p], isem[p]).wait()

  ids_start(0, 0, 0)
  for f in range(F):
    pltpu.sync_copy(tab_hbm.at[f, c], col_v)
    for j in range(NIDC):
      p = j % 2
      ids_wait(f, j, p)
      if j + 1 < NIDC:
        ids_start(f, j + 1, 1 - p)
      elif f + 1 < F:
        ids_start(f + 1, 0, 1 - p)
      idsb = ids_v[p]
      jbase = j * IDS_CHUNK

      @plsc.parallel_loop(0, IDS_CHUNK, step=LANES, unroll=8)
      def blk_body(off, f=f, idsb=idsb, jbase=jbase):
        vidx = idsb[pl.ds(off, LANES)]
        vals = plsc.load_gather(col_v, [vidx])
        sl = pl.ds(jbase + off, LANES)
        if f == 0:
          acc_v[sl] = vals
        else:
          plsc.addupdate(acc_v.at[sl], vals)

  pltpu.sync_copy(acc_v, out_hbm.at[c])


@jax.jit
def kernel(ids, tables):
  ids_t = ids.T
  tab_t = tables.transpose(0, 2, 1)

  mesh = plsc.VectorSubcoreMesh(core_axis_name="c", subcore_axis_name="s")
  run = pl.kernel(
      _body,
      out_type=jax.ShapeDtypeStruct((D, B), jnp.float32),
      mesh=mesh,
      scratch_types=[
          pltpu.VMEM((V,), jnp.float32),
          pltpu.VMEM((IDS_CHUNK,), jnp.int32),
          pltpu.VMEM((IDS_CHUNK,), jnp.int32),
          pltpu.VMEM((B,), jnp.float32),
          pltpu.SemaphoreType.DMA,
          pltpu.SemaphoreType.DMA,
          pltpu.SemaphoreType.DMA,
      ],
      compiler_params=pltpu.CompilerParams(needs_layout_passes=False),
  )
  return run(ids_t, tab_t).T
